# Initial kernel scaffold; baseline (speedup 1.0000x reference)
import jax
import jax.numpy as jnp
from jax.experimental import pallas as pl


def kernel(x, edge_index, W1, b1, W2, b2):
    raise NotImplementedError("write your pallas kernel here")


# speedup vs baseline: 9.0341x; 9.0341x over previous
import functools

import jax
import jax.numpy as jnp
from jax import lax
from jax.experimental import pallas as pl
from jax.experimental.pallas import tpu as pltpu
from jax.experimental.pallas import tpu_sc as plsc

N = 10000
E = 320000
NPAD = 10240
DUMMY_DST = 10200
KE = 128
EP = 323584
NSUB = 16
NCORE = 2
RPT = NPAD // NSUB
TROWS1 = 2 * N + (NPAD - N)
TROWS2 = 2 * NPAD


def _mesh():
    return plsc.VectorSubcoreMesh(core_axis_name="c", subcore_axis_name="s")


@functools.partial(
    pl.kernel,
    out_type=jax.ShapeDtypeStruct((NCORE * NPAD, 128), jnp.float32),
    mesh=_mesh(),
    scratch_types=[
        pltpu.VMEM((KE,), jnp.int32),
        pltpu.VMEM((KE, 128), jnp.float32),
        pltpu.VMEM_SHARED((NPAD, 128), jnp.float32),
    ],
)
def _deg_kernel(dst_hbm, out_hbm, idx_v, rows_v, acc):
    c = lax.axis_index("c")
    s = lax.axis_index("s")

    def fill(i, val):
        for j in range(8):
            rows_v[i, pl.ds(16 * j, 16)] = jnp.full((16,), val, jnp.float32)
        return val

    lax.fori_loop(0, KE, fill, 0.0)
    for r in range(RPT // KE):
        pltpu.sync_copy(rows_v, acc.at[pl.ds(s * RPT + r * KE, KE)])
    lax.fori_loop(0, KE, fill, 1.0)
    plsc.subcore_barrier()

    ept = EP // (NCORE * NSUB)
    nch = ept // KE

    def body(i, carry):
        base = pl.multiple_of((c * NSUB + s) * ept + i * KE, 8)
        pltpu.sync_copy(dst_hbm.at[pl.ds(base, KE)], idx_v)
        pltpu.sync_copy(rows_v, acc.at[idx_v], add=True)
        return carry

    lax.fori_loop(0, nch, body, 0)
    plsc.subcore_barrier()
    obase = pl.multiple_of(c * NPAD + s * RPT, 8)
    pltpu.sync_copy(acc.at[pl.ds(s * RPT, RPT)], out_hbm.at[pl.ds(obase, RPT)])


def _make_scatter(ept, half):
    nch = ept // KE

    @functools.partial(
        pl.kernel,
        out_type=jax.ShapeDtypeStruct((NCORE * NPAD, 128), jnp.float32),
        mesh=_mesh(),
        scratch_types=[
            pltpu.VMEM((KE,), jnp.int32),
            pltpu.VMEM((KE,), jnp.int32),
            pltpu.VMEM((KE, 128), jnp.float32),
            pltpu.VMEM_SHARED((NPAD, 128), jnp.float32),
            pltpu.SemaphoreType.DMA,
        ],
    )
    def _scatter(table_hbm, src_hbm, dst_hbm, out_hbm, src_v, dst_v, rows_v, acc, sem):
        c = lax.axis_index("c")
        s = lax.axis_index("s")
        tbase = pl.multiple_of(c * half + s * RPT, 8)
        pltpu.sync_copy(table_hbm.at[pl.ds(tbase, RPT)], acc.at[pl.ds(s * RPT, RPT)])
        plsc.subcore_barrier()

        def body(i, carry):
            base = pl.multiple_of((c * NSUB + s) * ept + i * KE, 8)
            pltpu.sync_copy(src_hbm.at[pl.ds(base, KE)], src_v)
            dbase = pl.multiple_of((c * NSUB + s) * ept % EP + i * KE, 8)
            pltpu.sync_copy(dst_hbm.at[pl.ds(dbase, KE)], dst_v)
            pltpu.async_copy(table_hbm.at[src_v], rows_v, sem).wait()
            pltpu.sync_copy(rows_v, acc.at[dst_v], add=True)
            return carry

        lax.fori_loop(0, nch, body, 0)
        plsc.subcore_barrier()
        obase = pl.multiple_of(c * NPAD + s * RPT, 8)
        pltpu.sync_copy(acc.at[pl.ds(s * RPT, RPT)], out_hbm.at[pl.ds(obase, RPT)])

    return _scatter


_scatter_l1 = _make_scatter(EP // NSUB, N)
_scatter_l2 = _make_scatter(EP // (NCORE * NSUB), NPAD)


def _mm(a, w, bm=1000):
    m, k = a.shape
    _, n = w.shape

    def body(a_ref, w_ref, o_ref):
        o_ref[...] = jnp.dot(a_ref[...], w_ref[...],
                             preferred_element_type=jnp.float32)

    return pl.pallas_call(
        body,
        grid=(m // bm,),
        in_specs=[pl.BlockSpec((bm, k), lambda i: (i, 0)),
                  pl.BlockSpec((k, n), lambda i: (0, 0))],
        out_specs=pl.BlockSpec((bm, n), lambda i: (i, 0)),
        out_shape=jax.ShapeDtypeStruct((m, n), jnp.float32),
    )(a, w)


def kernel(x, edge_index, W1, b1, W2, b2):
    src = edge_index[0].astype(jnp.int32)
    dst = edge_index[1].astype(jnp.int32)
    npad_e = EP - E
    srcs1 = jnp.concatenate([
        src, jnp.full((npad_e,), 2 * N, jnp.int32),
        src + N, jnp.full((npad_e,), 2 * N, jnp.int32),
    ])
    srcs2 = jnp.concatenate([src, jnp.full((npad_e,), N, jnp.int32)])
    dstp = jnp.concatenate([dst, jnp.full((npad_e,), DUMMY_DST, jnp.int32)])

    degp = _deg_kernel(dstp)
    deg = degp[:N, 0] + degp[NPAD:NPAD + N, 0] + 1.0
    dinv = lax.rsqrt(deg)

    t1 = _mm(x, W1) * dinv[:, None]
    table1 = jnp.concatenate(
        [t1[:, :128], t1[:, 128:], jnp.zeros((TROWS1 - 2 * N, 128), jnp.float32)],
        axis=0)
    s1 = _scatter_l1(table1, srcs1, dstp)
    s1f = jnp.concatenate([s1[:N], s1[NPAD:NPAD + N]], axis=1)
    a1 = jax.nn.relu(s1f * dinv[:, None] + b1)

    t2 = _mm(a1, W2) * dinv[:, None]
    table2 = jnp.concatenate(
        [t2, jnp.zeros((TROWS2 - N, 128), jnp.float32)], axis=0)
    s2 = _scatter_l2(table2, srcs2, dstp)
    s2f = s2[:N] + s2[NPAD:NPAD + N]
    return jax.nn.relu(s2f * dinv[:, None] + b2)


# speedup vs baseline: 9.5159x; 1.0533x over previous
import functools

import jax
import jax.numpy as jnp
from jax import lax
from jax.experimental import pallas as pl
from jax.experimental.pallas import tpu as pltpu
from jax.experimental.pallas import tpu_sc as plsc

N = 10000
E = 320000
NPAD = 10112
DUMMY_DST = 10050
KE = 128
EP = 327680
NSUB = 16
NCORE = 2
NT = NCORE * NSUB
RPT = NPAD // NSUB
NCH1 = EP // (NSUB * KE)
NCH2 = EP // (NT * KE)
BLK = 16
NBUF = 2
TROWS1 = 2 * N + (NPAD - N)
TROWS2 = 2 * NPAD


def _mesh():
    return plsc.VectorSubcoreMesh(core_axis_name="c", subcore_axis_name="s")


@functools.partial(
    pl.kernel,
    out_type=jax.ShapeDtypeStruct((NCORE * NPAD, 128), jnp.float32),
    mesh=_mesh(),
    scratch_types=[
        pltpu.VMEM((NCH2, 2, KE), jnp.int32),
        pltpu.VMEM((KE, 128), jnp.float32),
        pltpu.VMEM_SHARED((NPAD, 128), jnp.float32),
        pltpu.SemaphoreType.DMA,
    ],
)
def _deg_kernel(idx_hbm, out_hbm, idx_v, rows_v, acc, sem):
    c = lax.axis_index("c")
    s = lax.axis_index("s")
    tid = c * NSUB + s
    pltpu.sync_copy(idx_hbm.at[tid], idx_v)

    def fill(i, val):
        for j in range(8):
            rows_v[i, pl.ds(16 * j, 16)] = jnp.full((16,), val, jnp.float32)
        return val

    lax.fori_loop(0, KE, fill, 0.0)
    for r in range(RPT // KE):
        pltpu.sync_copy(rows_v, acc.at[pl.ds(s * RPT + r * KE, KE)])
    rem = RPT % KE
    if rem:
        pltpu.sync_copy(rows_v.at[pl.ds(0, rem)],
                        acc.at[pl.ds(s * RPT + (RPT // KE) * KE, rem)])
    lax.fori_loop(0, KE, fill, 1.0)
    plsc.subcore_barrier()

    def body(i, carry):
        pltpu.async_copy(rows_v, acc.at[idx_v.at[i, 1]], sem, add=True)
        return carry

    lax.fori_loop(0, NCH2, body, 0)

    def drain(i, carry):
        pltpu.make_async_copy(out_hbm.at[pl.ds(0, KE)], rows_v, sem).wait()
        return carry

    lax.fori_loop(0, NCH2, drain, 0)
    plsc.subcore_barrier()
    obase = pl.multiple_of(c * NPAD + s * RPT, 8)
    pltpu.sync_copy(acc.at[pl.ds(s * RPT, RPT)], out_hbm.at[pl.ds(obase, RPT)])


def _make_scatter(nch, half):
    @functools.partial(
        pl.kernel,
        out_type=jax.ShapeDtypeStruct((NCORE * NPAD, 128), jnp.float32),
        mesh=_mesh(),
        scratch_types=[
            pltpu.VMEM((BLK, 2, KE), jnp.int32),
            pltpu.VMEM((KE, 128), jnp.float32),
            pltpu.VMEM((KE, 128), jnp.float32),
            pltpu.VMEM_SHARED((NPAD, 128), jnp.float32),
            pltpu.SemaphoreType.DMA,
            pltpu.SemaphoreType.DMA,
        ],
    )
    def _scatter(table_hbm, idx_hbm, out_hbm, idx_v, r0, r1, acc, sem0, sem1):
        rows = (r0, r1)
        sems = (sem0, sem1)
        c = lax.axis_index("c")
        s = lax.axis_index("s")
        tid = c * NSUB + s
        tbase = pl.multiple_of(c * half + s * RPT, 8)
        pltpu.sync_copy(table_hbm.at[pl.ds(tbase, RPT)],
                        acc.at[pl.ds(s * RPT, RPT)])
        plsc.subcore_barrier()

        def body(blk, carry):
            pltpu.sync_copy(idx_hbm.at[tid, pl.ds(blk * BLK, BLK)], idx_v)
            for b in range(NBUF):
                pltpu.async_copy(table_hbm.at[idx_v.at[b, 0]], rows[b], sems[b])
            for j in range(BLK):
                b = j % NBUF
                pltpu.make_async_copy(table_hbm.at[pl.ds(0, KE)], rows[b],
                                      sems[b]).wait()
                pltpu.sync_copy(rows[b], acc.at[idx_v.at[j, 1]], add=True)
                if j + NBUF < BLK:
                    pltpu.async_copy(table_hbm.at[idx_v.at[j + NBUF, 0]],
                                     rows[b], sems[b])
            return carry

        lax.fori_loop(0, nch // BLK, body, 0)
        plsc.subcore_barrier()
        obase = pl.multiple_of(c * NPAD + s * RPT, 8)
        pltpu.sync_copy(acc.at[pl.ds(s * RPT, RPT)],
                        out_hbm.at[pl.ds(obase, RPT)])

    return _scatter


_scatter_l1 = _make_scatter(NCH1, N)
_scatter_l2 = _make_scatter(NCH2, NPAD)


def _mm(a, w, bm=1000):
    m, k = a.shape
    _, n = w.shape

    def body(a_ref, w_ref, o_ref):
        o_ref[...] = jnp.dot(a_ref[...], w_ref[...],
                             preferred_element_type=jnp.float32)

    return pl.pallas_call(
        body,
        grid=(m // bm,),
        in_specs=[pl.BlockSpec((bm, k), lambda i: (i, 0)),
                  pl.BlockSpec((k, n), lambda i: (0, 0))],
        out_specs=pl.BlockSpec((bm, n), lambda i: (i, 0)),
        out_shape=jax.ShapeDtypeStruct((m, n), jnp.float32),
    )(a, w)


def kernel(x, edge_index, W1, b1, W2, b2):
    src = edge_index[0].astype(jnp.int32)
    dst = edge_index[1].astype(jnp.int32)
    npad_e = EP - E
    src1 = jnp.concatenate([src, jnp.full((npad_e,), 2 * N, jnp.int32)])
    srcs1 = jnp.concatenate([src1, src1 + N]).reshape(NT, NCH1, KE)
    srcs2 = jnp.concatenate(
        [src, jnp.full((npad_e,), N, jnp.int32)]).reshape(NT, NCH2, KE)
    dstp = jnp.concatenate([dst, jnp.full((npad_e,), DUMMY_DST, jnp.int32)])
    dst1 = jnp.concatenate([dstp, dstp]).reshape(NT, NCH1, KE)
    dst2 = dstp.reshape(NT, NCH2, KE)
    idx1 = jnp.stack([srcs1, dst1], axis=2)
    idx2 = jnp.stack([srcs2, dst2], axis=2)

    degp = _deg_kernel(idx2)
    deg = degp[:N, 0] + degp[NPAD:NPAD + N, 0] + 1.0
    dinv = lax.rsqrt(deg)

    t1 = _mm(x, W1) * dinv[:, None]
    table1 = jnp.concatenate(
        [t1[:, :128], t1[:, 128:], jnp.zeros((TROWS1 - 2 * N, 128), jnp.float32)],
        axis=0)
    s1 = _scatter_l1(table1, idx1)
    s1f = jnp.concatenate([s1[:N], s1[NPAD:NPAD + N]], axis=1)
    a1 = jax.nn.relu(s1f * dinv[:, None] + b1)

    t2 = _mm(a1, W2) * dinv[:, None]
    table2 = jnp.concatenate(
        [t2, jnp.zeros((TROWS2 - N, 128), jnp.float32)], axis=0)
    s2 = _scatter_l2(table2, idx2)
    s2f = s2[:N] + s2[NPAD:NPAD + N]
    return jax.nn.relu(s2f * dinv[:, None] + b2)


# speedup vs baseline: 9.5345x; 1.0020x over previous
import functools

import jax
import jax.numpy as jnp
from jax import lax
from jax.experimental import pallas as pl
from jax.experimental.pallas import tpu as pltpu
from jax.experimental.pallas import tpu_sc as plsc

N = 10000
E = 320000
NPAD = 10112
DUMMY_DST = 10050
KE = 128
EP = 327680
NSUB = 16
NCORE = 2
NT = NCORE * NSUB
RPT = NPAD // NSUB
NCH1 = EP // (NSUB * KE)
NCH2 = EP // (NT * KE)
BLK = 16
NBUF = 2
TROWS1 = 2 * N + (NPAD - N)
TROWS2 = N + NPAD // NSUB + 8


def _mesh():
    return plsc.VectorSubcoreMesh(core_axis_name="c", subcore_axis_name="s")


@functools.partial(
    pl.kernel,
    out_type=jax.ShapeDtypeStruct((NCORE * NPAD, 128), jnp.float32),
    mesh=_mesh(),
    scratch_types=[
        pltpu.VMEM((NCH2, 2, KE), jnp.int32),
        pltpu.VMEM((KE, 128), jnp.float32),
        pltpu.VMEM_SHARED((NPAD, 128), jnp.float32),
        pltpu.SemaphoreType.DMA,
    ],
)
def _deg_kernel(idx_hbm, out_hbm, idx_v, rows_v, acc, sem):
    c = lax.axis_index("c")
    s = lax.axis_index("s")
    tid = c * NSUB + s
    pltpu.sync_copy(idx_hbm.at[tid], idx_v)

    def fill(i, val):
        for j in range(8):
            rows_v[i, pl.ds(16 * j, 16)] = jnp.full((16,), val, jnp.float32)
        return val

    lax.fori_loop(0, KE, fill, 0.0)
    for r in range(RPT // KE):
        pltpu.sync_copy(rows_v, acc.at[pl.ds(s * RPT + r * KE, KE)])
    rem = RPT % KE
    if rem:
        pltpu.sync_copy(rows_v.at[pl.ds(0, rem)],
                        acc.at[pl.ds(s * RPT + (RPT // KE) * KE, rem)])
    lax.fori_loop(0, KE, fill, 1.0)
    plsc.subcore_barrier()

    def body(i, carry):
        pltpu.async_copy(rows_v, acc.at[idx_v.at[i, 1]], sem, add=True)
        return carry

    lax.fori_loop(0, NCH2, body, 0)

    def drain(i, carry):
        pltpu.make_async_copy(out_hbm.at[pl.ds(0, KE)], rows_v, sem).wait()
        return carry

    lax.fori_loop(0, NCH2, drain, 0)
    plsc.subcore_barrier()
    obase = pl.multiple_of(c * NPAD + s * RPT, 8)
    pltpu.sync_copy(acc.at[pl.ds(s * RPT, RPT)], out_hbm.at[pl.ds(obase, RPT)])


def _make_scatter(nch, half):
    @functools.partial(
        pl.kernel,
        out_type=jax.ShapeDtypeStruct((NCORE * NPAD, 128), jnp.float32),
        mesh=_mesh(),
        scratch_types=[
            pltpu.VMEM((BLK, 2, KE), jnp.int32),
            pltpu.VMEM((KE, 128), jnp.float32),
            pltpu.VMEM((KE, 128), jnp.float32),
            pltpu.VMEM_SHARED((NPAD, 128), jnp.float32),
            pltpu.SemaphoreType.DMA,
            pltpu.SemaphoreType.DMA,
        ],
    )
    def _scatter(table_hbm, idx_hbm, out_hbm, idx_v, r0, r1, acc, sem0, sem1):
        rows = (r0, r1)
        sems = (sem0, sem1)
        c = lax.axis_index("c")
        s = lax.axis_index("s")
        tid = c * NSUB + s
        if half is None:
            tbase = pl.multiple_of((1 - c) * (s * RPT) + c * N, 8)
        else:
            tbase = pl.multiple_of(c * half + s * RPT, 8)
        pltpu.sync_copy(table_hbm.at[pl.ds(tbase, RPT)],
                        acc.at[pl.ds(s * RPT, RPT)])
        plsc.subcore_barrier()

        def body(blk, carry):
            pltpu.sync_copy(idx_hbm.at[tid, pl.ds(blk * BLK, BLK)], idx_v)
            for b in range(NBUF):
                pltpu.async_copy(table_hbm.at[idx_v.at[b, 0]], rows[b], sems[b])
            for j in range(BLK):
                b = j % NBUF
                pltpu.make_async_copy(table_hbm.at[pl.ds(0, KE)], rows[b],
                                      sems[b]).wait()
                pltpu.sync_copy(rows[b], acc.at[idx_v.at[j, 1]], add=True)
                if j + NBUF < BLK:
                    pltpu.async_copy(table_hbm.at[idx_v.at[j + NBUF, 0]],
                                     rows[b], sems[b])
            return carry

        lax.fori_loop(0, nch // BLK, body, 0)
        plsc.subcore_barrier()
        obase = pl.multiple_of(c * NPAD + s * RPT, 8)
        pltpu.sync_copy(acc.at[pl.ds(s * RPT, RPT)],
                        out_hbm.at[pl.ds(obase, RPT)])

    return _scatter


_scatter_l1 = _make_scatter(NCH1, N)
_scatter_l2 = _make_scatter(NCH2, NPAD)


def _mm(a, w, bm=1000):
    m, k = a.shape
    _, n = w.shape

    def body(a_ref, w_ref, o_ref):
        o_ref[...] = jnp.dot(a_ref[...], w_ref[...],
                             preferred_element_type=jnp.float32)

    return pl.pallas_call(
        body,
        grid=(m // bm,),
        in_specs=[pl.BlockSpec((bm, k), lambda i: (i, 0)),
                  pl.BlockSpec((k, n), lambda i: (0, 0))],
        out_specs=pl.BlockSpec((bm, n), lambda i: (i, 0)),
        out_shape=jax.ShapeDtypeStruct((m, n), jnp.float32),
    )(a, w)


def kernel(x, edge_index, W1, b1, W2, b2):
    src = edge_index[0].astype(jnp.int32)
    dst = edge_index[1].astype(jnp.int32)
    npad_e = EP - E
    src1 = jnp.concatenate([src, jnp.full((npad_e,), 2 * N, jnp.int32)])
    srcs1 = jnp.concatenate([src1, src1 + N]).reshape(NT, NCH1, KE)
    srcs2 = jnp.concatenate(
        [src, jnp.full((npad_e,), N, jnp.int32)]).reshape(NT, NCH2, KE)
    dstp = jnp.concatenate([dst, jnp.full((npad_e,), DUMMY_DST, jnp.int32)])
    dst1 = jnp.concatenate([dstp, dstp]).reshape(NT, NCH1, KE)
    dst2 = dstp.reshape(NT, NCH2, KE)
    idx1 = jnp.stack([srcs1, dst1], axis=2)
    idx2 = jnp.stack([srcs2, dst2], axis=2)

    degp = _deg_kernel(idx2)
    deg = degp[:N, 0] + degp[NPAD:NPAD + N, 0] + 1.0
    dinv = lax.rsqrt(deg)

    t1 = _mm(x, W1) * dinv[:, None]
    table1 = jnp.concatenate(
        [t1[:, :128], t1[:, 128:], jnp.zeros((TROWS1 - 2 * N, 128), jnp.float32)],
        axis=0)
    s1 = _scatter_l1(table1, idx1)
    s1f = jnp.concatenate([s1[:N], s1[NPAD:NPAD + N]], axis=1)
    a1 = jax.nn.relu(s1f * dinv[:, None] + b1)

    t2 = _mm(a1, W2) * dinv[:, None]
    table2 = jnp.concatenate(
        [t2, jnp.zeros((TROWS2 - N, 128), jnp.float32)], axis=0)
    s2 = _scatter_l2(table2, idx2)
    s2f = s2[:N] + s2[NPAD:NPAD + N]
    return jax.nn.relu(s2f * dinv[:, None] + b2)
